# Initial kernel scaffold; baseline (speedup 1.0000x reference)
import jax
import jax.numpy as jnp
from jax.experimental import pallas as pl


def kernel(x, edge_index, Wih, Whh, bih, bhh, Wl, bl, Wr, br, att, b_gat, Wp, bp):
    raise NotImplementedError("write your pallas kernel here")


# speedup vs baseline: 14.5219x; 14.5219x over previous
import functools

import jax
import jax.numpy as jnp
from jax import lax
from jax.experimental import pallas as pl
from jax.experimental.pallas import tpu as pltpu
from jax.experimental.pallas import tpu_sc as plsc

CH = 64

_SC_PARAMS = pltpu.CompilerParams(needs_layout_passes=False,
                                  use_tc_tiling_on_sc=False)


def _lstm_body(T, F, H, x_ref, wih_ref, whh_ref, b_ref, wl_ref, bl_ref,
               wr_ref, br_ref, xlr_ref):
    xb = x_ref[...]
    B = xb.shape[0]
    h = jnp.zeros((B, H), jnp.float32)
    c = jnp.zeros((B, H), jnp.float32)
    wih = wih_ref[...]
    whh = whh_ref[...]
    b = b_ref[...]
    for t in range(T):
        xi = xb[:, t * F:(t + 1) * F]
        g = jnp.dot(xi, wih, preferred_element_type=jnp.float32)
        g = g + jnp.dot(h, whh, preferred_element_type=jnp.float32) + b
        i = jax.nn.sigmoid(g[:, :H])
        f = jax.nn.sigmoid(g[:, H:2 * H])
        gg = jnp.tanh(g[:, 2 * H:3 * H])
        o = jax.nn.sigmoid(g[:, 3 * H:])
        c = f * c + i * gg
        h = o * jnp.tanh(c)
    HH = wl_ref.shape[1]
    xlr_ref[:, :HH] = jnp.dot(h, wl_ref[...],
                              preferred_element_type=jnp.float32) + bl_ref[...]
    xlr_ref[:, HH:] = jnp.dot(h, wr_ref[...],
                              preferred_element_type=jnp.float32) + br_ref[...]


def _lstm_proj(x2, wihT, whhT, b2, wlT, bl2, wrT, br2):
    N, TF = x2.shape
    H = whhT.shape[0]
    HH = wlT.shape[1]
    BN = 2000
    grid = N // BN
    T = TF // wihT.shape[0]
    F = wihT.shape[0]
    return pl.pallas_call(
        functools.partial(_lstm_body, T, F, H),
        grid=(grid,),
        in_specs=[
            pl.BlockSpec((BN, TF), lambda i: (i, 0)),
            pl.BlockSpec(wihT.shape, lambda i: (0, 0)),
            pl.BlockSpec(whhT.shape, lambda i: (0, 0)),
            pl.BlockSpec(b2.shape, lambda i: (0, 0)),
            pl.BlockSpec(wlT.shape, lambda i: (0, 0)),
            pl.BlockSpec(bl2.shape, lambda i: (0, 0)),
            pl.BlockSpec(wrT.shape, lambda i: (0, 0)),
            pl.BlockSpec(br2.shape, lambda i: (0, 0)),
        ],
        out_specs=pl.BlockSpec((BN, 2 * HH), lambda i: (i, 0)),
        out_shape=jax.ShapeDtypeStruct((N, 2 * HH), jnp.float32),
    )(x2, wihT, whhT, b2, wlT, bl2, wrT, br2)


def _sc_pass1(xlr, src, dst, att_tab, z1):
    N, HH2 = xlr.shape
    HH = HH2 // 2
    C = HH // 2
    E = src.shape[0]
    NR = z1.shape[0]
    nch = E // CH
    base_c = nch // 32
    extra = nch % 32
    mesh = plsc.VectorSubcoreMesh(core_axis_name="c", subcore_axis_name="s")

    @functools.partial(
        pl.kernel,
        mesh=mesh,
        compiler_params=_SC_PARAMS,
        out_type=[
            jax.ShapeDtypeStruct((E,), jnp.float32),
            jax.ShapeDtypeStruct((E,), jnp.float32),
            jax.ShapeDtypeStruct((2 * NR, 128), jnp.float32),
        ],
        scratch_types=[
            pltpu.VMEM((CH,), jnp.int32),
            pltpu.VMEM((CH,), jnp.int32),
            pltpu.VMEM((CH, HH2), jnp.float32),
            pltpu.VMEM((CH, HH2), jnp.float32),
            pltpu.VMEM((CH,), jnp.float32),
            pltpu.VMEM((CH,), jnp.float32),
            pltpu.VMEM((16 * HH,), jnp.float32),
            pltpu.VMEM((NR, 128), jnp.float32),
            pltpu.VMEM((80,), jnp.int32),
            pltpu.VMEM_SHARED((NR, 128), jnp.float32),
            pltpu.SemaphoreType.DMA,
        ],
    )
    def k(xlr_hbm, src_hbm, dst_hbm, att_hbm, z1_hbm,
          ea0_hbm, ea1_hbm, asum_hbm,
          src_v, dst_v, glv, grv, ea0v, ea1v, att_v, asumv, ridx, asum_sp,
          sem):
        cid = lax.axis_index("c")
        sid = lax.axis_index("s")
        wid = sid * 2 + cid
        nc = base_c + jnp.where(wid < extra, 1, 0)
        start = wid * base_c + jnp.minimum(wid, extra)

        pltpu.sync_copy(att_hbm, att_v)

        @pl.when(sid == 0)
        def _():
            pltpu.sync_copy(z1_hbm, asum_sp)

        zf16 = jnp.zeros((16,), jnp.float32)
        iota16 = lax.iota(jnp.int32, 16)

        def zero_body(i, _):
            li = iota16 + i * 16
            plsc.store_scatter(asumv, [li >> 7, li & 127], zf16)
            return 0

        lax.fori_loop(0, NR * 8, zero_body, 0)

        def chunk_body(ci, _):
            cb = (start + ci) * CH
            pltpu.sync_copy(src_hbm.at[pl.ds(cb, CH)], src_v)
            pltpu.sync_copy(dst_hbm.at[pl.ds(cb, CH)], dst_v)
            pltpu.async_copy(xlr_hbm.at[src_v], glv, sem).wait()
            pltpu.async_copy(xlr_hbm.at[dst_v], grv, sem).wait()

            def group_body(g, _):
                li = lax.iota(jnp.int32, 16) + g * 16
                acc0 = jnp.zeros((16,), jnp.float32)
                acc1 = jnp.zeros((16,), jnp.float32)
                for c in range(HH):
                    cvec = jnp.full((16,), c, jnp.int32)
                    rvec = jnp.full((16,), c + HH, jnp.int32)
                    s = (plsc.load_gather(glv, [li, cvec]) +
                         plsc.load_gather(grv, [li, rvec]))
                    m = jnp.maximum(s, 0.2 * s)
                    ma = m * att_v[pl.ds(c * 16, 16)]
                    if c < C:
                        acc0 = acc0 + ma
                    else:
                        acc1 = acc1 + ma
                e0 = jnp.exp(acc0)
                e1 = jnp.exp(acc1)
                plsc.store_scatter(ea0v, [li], e0)
                plsc.store_scatter(ea1v, [li], e1)
                d16 = plsc.load_gather(dst_v, [li])
                p0 = d16 * 2
                p1 = p0 + 1
                plsc.addupdate_scatter(asumv, [p0 >> 7, p0 & 127], e0)
                plsc.addupdate_scatter(asumv, [p1 >> 7, p1 & 127], e1)
                return 0

            lax.fori_loop(0, CH // 16, group_body, 0)
            pltpu.sync_copy(ea0v, ea0_hbm.at[pl.ds(cb, CH)])
            pltpu.sync_copy(ea1v, ea1_hbm.at[pl.ds(cb, CH)])
            return 0

        lax.fori_loop(0, nc, chunk_body, 0)
        plsc.subcore_barrier()

        def merge_body(b, _):
            rb = b * 80
            for kk in range(5):
                plsc.store_scatter(ridx, [iota16 + kk * 16],
                                   iota16 + (rb + kk * 16))
            pltpu.sync_copy(asumv.at[pl.ds(rb, 80)], asum_sp.at[ridx],
                            add=True)
            return 0

        lax.fori_loop(0, NR // 80, merge_body, 0)
        plsc.subcore_barrier()

        @pl.when(sid == 0)
        def _():
            pltpu.sync_copy(asum_sp, asum_hbm.at[pl.ds(cid * NR, NR)])

    return k(xlr, src, dst, att_tab, z1)


def _sc_pass2(xlr, src, dst, ea0, ea1, asum_part, z32):
    N, HH2 = xlr.shape
    C = HH2 // 4
    E = src.shape[0]
    NR = asum_part.shape[0] // 2
    SB = NR // 16
    nch = E // CH
    base_c = nch // 32
    extra = nch % 32
    mesh = plsc.VectorSubcoreMesh(core_axis_name="c", subcore_axis_name="s")

    @functools.partial(
        pl.kernel,
        mesh=mesh,
        compiler_params=_SC_PARAMS,
        out_type=jax.ShapeDtypeStruct((2, N, C), jnp.float32),
        scratch_types=[
            pltpu.VMEM((CH,), jnp.int32),
            pltpu.VMEM((CH,), jnp.int32),
            pltpu.VMEM((CH,), jnp.int32),
            pltpu.VMEM((CH, HH2), jnp.float32),
            pltpu.VMEM((CH,), jnp.float32),
            pltpu.VMEM((CH,), jnp.float32),
            pltpu.VMEM((CH, 128), jnp.float32),
            pltpu.VMEM((CH, 128), jnp.float32),
            pltpu.VMEM((CH, C), jnp.float32),
            pltpu.VMEM_SHARED((N, C), jnp.float32),
            pltpu.SemaphoreType.DMA,
        ],
    )
    def k(xlr_hbm, src_hbm, dst_hbm, ea0_hbm, ea1_hbm, asum_hbm, z32_hbm,
          acc_hbm,
          src_v, dst_v, rowv, glv, ea0v, ea1v, ab0, ab1, msgv, acc_sp, sem):
        cid = lax.axis_index("c")
        sid = lax.axis_index("s")
        wid = sid * 2 + cid
        nc = base_c + jnp.where(wid < extra, 1, 0)
        start = wid * base_c + jnp.minimum(wid, extra)

        @pl.when(sid == 0)
        def _():
            pltpu.sync_copy(z32_hbm, acc_sp)

        plsc.subcore_barrier()
        iota16 = lax.iota(jnp.int32, 16)

        def chunk_body(ci, _):
            cb = (start + ci) * CH
            pltpu.sync_copy(src_hbm.at[pl.ds(cb, CH)], src_v)
            pltpu.sync_copy(dst_hbm.at[pl.ds(cb, CH)], dst_v)
            pltpu.sync_copy(ea0_hbm.at[pl.ds(cb, CH)], ea0v)
            pltpu.sync_copy(ea1_hbm.at[pl.ds(cb, CH)], ea1v)

            def row_body(g, _):
                li = iota16 + g * 16
                d16 = plsc.load_gather(dst_v, [li])
                plsc.store_scatter(rowv, [li], (d16 * 2) >> 7)
                return 0

            lax.fori_loop(0, CH // 16, row_body, 0)
            pltpu.async_copy(xlr_hbm.at[src_v], glv, sem).wait()
            pltpu.async_copy(asum_hbm.at[rowv], ab0, sem).wait()

            def row2_body(g, _):
                li = iota16 + g * 16
                r16 = plsc.load_gather(rowv, [li])
                plsc.store_scatter(rowv, [li], r16 + NR)
                return 0

            lax.fori_loop(0, CH // 16, row2_body, 0)
            pltpu.async_copy(asum_hbm.at[rowv], ab1, sem).wait()

            def group_body(g, _):
                li = iota16 + g * 16
                e0 = plsc.load_gather(ea0v, [li])
                e1 = plsc.load_gather(ea1v, [li])
                d16 = plsc.load_gather(dst_v, [li])
                col = (d16 * 2) & 127
                a0 = (plsc.load_gather(ab0, [li, col]) +
                      plsc.load_gather(ab1, [li, col]))
                a1 = (plsc.load_gather(ab0, [li, col + 1]) +
                      plsc.load_gather(ab1, [li, col + 1]))
                w0 = 0.5 * e0 / (a0 + 1e-16)
                w1 = 0.5 * e1 / (a1 + 1e-16)
                for c in range(C):
                    cvec = jnp.full((16,), c, jnp.int32)
                    cvec2 = jnp.full((16,), c + C, jnp.int32)
                    x0 = plsc.load_gather(glv, [li, cvec])
                    x1 = plsc.load_gather(glv, [li, cvec2])
                    plsc.store_scatter(msgv, [li, cvec], x0 * w0 + x1 * w1)
                return 0

            lax.fori_loop(0, CH // 16, group_body, 0)
            pltpu.sync_copy(msgv, acc_sp.at[dst_v], add=True)
            return 0

        lax.fori_loop(0, nc, chunk_body, 0)
        plsc.subcore_barrier()

        @pl.when(sid == 0)
        def _():
            pltpu.sync_copy(acc_sp, acc_hbm.at[cid])

    return k(xlr, src, dst, ea0, ea1, asum_part, z32)


def _final_body(acc_ref, bg_ref, wp_ref, bp_ref, o_ref):
    a = acc_ref[0] + acc_ref[1] + bg_ref[...]
    o = jnp.where(a > 0, a, jnp.exp(a) - 1.0)
    o_ref[...] = jnp.dot(o, wp_ref[...],
                         preferred_element_type=jnp.float32) + bp_ref[...]


def _final(acc_part, bg2, wpT, bp2):
    _, N, C = acc_part.shape
    BN = 10000
    return pl.pallas_call(
        _final_body,
        grid=(N // BN,),
        in_specs=[
            pl.BlockSpec((2, BN, C), lambda i: (0, i, 0)),
            pl.BlockSpec((1, C), lambda i: (0, 0)),
            pl.BlockSpec((C, 1), lambda i: (0, 0)),
            pl.BlockSpec((1, 1), lambda i: (0, 0)),
        ],
        out_specs=pl.BlockSpec((BN, 1), lambda i: (i, 0)),
        out_shape=jax.ShapeDtypeStruct((N, 1), jnp.float32),
    )(acc_part, bg2, wpT, bp2)


def kernel(x, edge_index, Wih, Whh, bih, bhh, Wl, bl, Wr, br, att, b_gat,
           Wp, bp):
    N, T, F = x.shape
    H = Whh.shape[1]
    HH = Wl.shape[0]

    x2 = x.reshape(N, T * F)
    b2 = (bih + bhh)[None, :]
    xlr = _lstm_proj(x2, Wih.T, Whh.T, b2, Wl.T, bl[None, :],
                     Wr.T, br[None, :])

    src = edge_index[0]
    dst = edge_index[1]
    att_tab = jnp.repeat(att.reshape(HH), 16)
    NR = -(-(2 * N) // 128)
    NR = -(-NR // 80) * 80
    z1 = jnp.zeros((NR, 128), jnp.float32)
    z32 = jnp.zeros((N, H), jnp.float32)

    ea0, ea1, asum_part = _sc_pass1(xlr, src, dst, att_tab, z1)
    acc_part = _sc_pass2(xlr, src, dst, ea0, ea1, asum_part, z32)
    pred = _final(acc_part, b_gat[None, :], Wp.T, bp[None, :])
    return pred[:, 0]
